# Initial kernel scaffold; baseline (speedup 1.0000x reference)
import jax
import jax.numpy as jnp
from jax.experimental import pallas as pl


def kernel(x, edge_index, edge_weight, params):
    raise NotImplementedError("write your pallas kernel here")


# speedup vs baseline: 33.9502x; 33.9502x over previous
import functools

import jax
import jax.numpy as jnp
from jax import lax
from jax.experimental import pallas as pl
from jax.experimental.pallas import tpu as pltpu
from jax.experimental.pallas import tpu_sc as plsc

N = 100000
E = 1600000
F_IN = 16
F_OUT = 32
HORIZON = 12

NP = 102400
NW = 32
NPS = NP // 16
EPW = E // NW
CH = 2000
NCHUNK = EPW // CH
GROUPS = CH // 16
CHP = 400
NCHUNKP = EPW // CHP
GROUPSP = CHP // 16

_f32 = jnp.float32
_i32 = jnp.int32

_mesh = plsc.VectorSubcoreMesh(core_axis_name="c", subcore_axis_name="s")


@functools.partial(
    pl.kernel,
    out_type=jax.ShapeDtypeStruct((2, NP), _f32),
    mesh=_mesh,
    scratch_types=[
        pltpu.VMEM((CH,), _i32),
        pltpu.VMEM((CH,), _i32),
        pltpu.VMEM((CH,), _f32),
        pltpu.VMEM((CH,), _f32),
        pltpu.VMEM((NPS,), _f32),
        pltpu.VMEM_SHARED((NP,), _f32),
    ],
)
def _deg_kernel(src_hbm, dst_hbm, ew_hbm, out_hbm,
                src_b, dst_b, ew_b, w_b, zbuf, deg_sh):
    cid = lax.axis_index("c")
    sid = lax.axis_index("s")
    wid = cid * 16 + sid
    rbase = sid * NPS

    zero16 = jnp.zeros((16,), _f32)

    def _z(i, _):
        zbuf[pl.ds(i * 16, 16)] = zero16
        return 0

    lax.fori_loop(0, NPS // 16, _z, 0, unroll=8)
    pltpu.sync_copy(zbuf, deg_sh.at[pl.ds(rbase, NPS)])
    plsc.subcore_barrier()

    ebase = wid * EPW

    def _chunk(ci, _):
        off = ebase + ci * CH
        pltpu.sync_copy(src_hbm.at[pl.ds(off, CH)], src_b)
        pltpu.sync_copy(dst_hbm.at[pl.ds(off, CH)], dst_b)
        pltpu.sync_copy(ew_hbm.at[pl.ds(off, CH)], ew_b)

        def _grp(g, _):
            s = src_b[pl.ds(g * 16, 16)]
            d = dst_b[pl.ds(g * 16, 16)]
            e = ew_b[pl.ds(g * 16, 16)]
            w_b[pl.ds(g * 16, 16)] = jnp.where(s == d, 0.0, e)
            return 0

        lax.fori_loop(0, GROUPS, _grp, 0, unroll=4)
        pltpu.sync_copy(w_b, deg_sh.at[src_b], add=True)
        return 0

    lax.fori_loop(0, NCHUNK, _chunk, 0)
    plsc.subcore_barrier()
    pltpu.sync_copy(deg_sh.at[pl.ds(rbase, NPS)],
                    out_hbm.at[cid, pl.ds(rbase, NPS)])


@functools.partial(
    pl.kernel,
    out_type=jax.ShapeDtypeStruct((2, NP, F_IN), _f32),
    mesh=_mesh,
    scratch_types=[
        pltpu.VMEM((CHP,), _i32),
        pltpu.VMEM((CHP,), _i32),
        pltpu.VMEM((CHP,), _f32),
        pltpu.VMEM((CHP,), _f32),
        pltpu.VMEM((CHP, F_IN), _f32),
        pltpu.VMEM_SHARED((NP, F_IN), _f32),
        pltpu.SemaphoreType.DMA,
    ],
    compiler_params=pltpu.CompilerParams(use_tc_tiling_on_sc=False),
)
def _prop_kernel(src_hbm, dst_hbm, ew_hbm, y_hbm, out_hbm,
                 src_b, dst_b, ew_b, w_b, rows, acc_sh, gsem):
    cid = lax.axis_index("c")
    sid = lax.axis_index("s")
    wid = cid * 16 + sid
    rbase = sid * NPS

    zero16 = jnp.zeros((16,), _f32)

    def _z(i, _):
        rows[i, :] = zero16
        return 0

    lax.fori_loop(0, CHP, _z, 0, unroll=8)
    for j in range(NPS // CHP):
        pltpu.sync_copy(rows, acc_sh.at[pl.ds(rbase + j * CHP, CHP)])
    plsc.subcore_barrier()

    ebase = wid * EPW

    def _chunk(ci, _):
        off = ebase + ci * CHP
        pltpu.sync_copy(src_hbm.at[pl.ds(off, CHP)], src_b)
        pltpu.sync_copy(dst_hbm.at[pl.ds(off, CHP)], dst_b)
        pltpu.sync_copy(ew_hbm.at[pl.ds(off, CHP)], ew_b)
        gather = pltpu.async_copy(y_hbm.at[src_b], rows, gsem)

        def _grp(g, _):
            s = src_b[pl.ds(g * 16, 16)]
            d = dst_b[pl.ds(g * 16, 16)]
            e = ew_b[pl.ds(g * 16, 16)]
            w_b[pl.ds(g * 16, 16)] = jnp.where(s == d, 0.0, -e)
            return 0

        lax.fori_loop(0, GROUPSP, _grp, 0, unroll=4)
        gather.wait()

        def _scale(g, _):
            w16 = w_b[pl.ds(g * 16, 16)]
            e0 = g * 16
            for j in range(16):
                wv = jnp.full((16,), w16[j], dtype=_f32)
                rows[e0 + j, :] = rows[e0 + j, :] * wv
            return 0

        lax.fori_loop(0, GROUPSP, _scale, 0, unroll=2)
        pltpu.sync_copy(rows, acc_sh.at[dst_b], add=True)
        return 0

    lax.fori_loop(0, NCHUNKP, _chunk, 0)
    plsc.subcore_barrier()
    pltpu.sync_copy(acc_sh.at[pl.ds(rbase, NPS)],
                    out_hbm.at[cid, pl.ds(rbase, NPS)])


_RB = 2048
_GRID = NP // _RB


def _y_body(deg_ref, x_ref, y_ref):
    deg = deg_ref[0] + deg_ref[1]
    dis = jnp.where(deg > 0, lax.rsqrt(deg), 0.0)
    y_ref[...] = dis * x_ref[...]


def _make_y(deg_pp, x_pad):
    return pl.pallas_call(
        _y_body,
        grid=(_GRID,),
        in_specs=[
            pl.BlockSpec((2, _RB, 1), lambda i: (0, i, 0)),
            pl.BlockSpec((_RB, F_IN), lambda i: (i, 0)),
        ],
        out_specs=pl.BlockSpec((_RB, F_IN), lambda i: (i, 0)),
        out_shape=jax.ShapeDtypeStruct((NP, F_IN), _f32),
    )(deg_pp.reshape(2, NP, 1), x_pad)


def _dense_body(x_ref, a_ref, deg_ref, w0_ref, w1_ref, bc_ref, wco_ref,
                wl_ref, bl_ref, h_ref, hh_ref, cc_ref):
    x = x_ref[...]
    a = a_ref[0] + a_ref[1]
    deg = deg_ref[0] + deg_ref[1]
    dis = jnp.where(deg > 0, lax.rsqrt(deg), 0.0)
    px = dis * a
    pre = (jnp.dot(x, w0_ref[...], preferred_element_type=_f32)
           + jnp.dot(px, w1_ref[...], preferred_element_type=_f32)
           + bc_ref[...])
    gi = jax.nn.sigmoid(pre[:, 0:F_OUT])
    gt = jnp.tanh(pre[:, F_OUT:2 * F_OUT])
    cc = gi * gt
    go = jax.nn.sigmoid(pre[:, 2 * F_OUT:3 * F_OUT] + wco_ref[...] * cc)
    hh = go * jnp.tanh(cc)
    h_ref[...] = (jnp.dot(jax.nn.relu(hh), wl_ref[...],
                          preferred_element_type=_f32) + bl_ref[...])
    hh_ref[...] = hh
    cc_ref[...] = cc


def _make_dense(x_pad, acc_pp, deg_pp, w0c, w1c, bc, wco, wl, bl):
    return pl.pallas_call(
        _dense_body,
        grid=(_GRID,),
        in_specs=[
            pl.BlockSpec((_RB, F_IN), lambda i: (i, 0)),
            pl.BlockSpec((2, _RB, F_IN), lambda i: (0, i, 0)),
            pl.BlockSpec((2, _RB, 1), lambda i: (0, i, 0)),
            pl.BlockSpec((F_IN, 3 * F_OUT), lambda i: (0, 0)),
            pl.BlockSpec((F_IN, 3 * F_OUT), lambda i: (0, 0)),
            pl.BlockSpec((1, 3 * F_OUT), lambda i: (0, 0)),
            pl.BlockSpec((1, F_OUT), lambda i: (0, 0)),
            pl.BlockSpec((F_OUT, HORIZON), lambda i: (0, 0)),
            pl.BlockSpec((1, HORIZON), lambda i: (0, 0)),
        ],
        out_specs=[
            pl.BlockSpec((_RB, HORIZON), lambda i: (i, 0)),
            pl.BlockSpec((_RB, F_OUT), lambda i: (i, 0)),
            pl.BlockSpec((_RB, F_OUT), lambda i: (i, 0)),
        ],
        out_shape=[
            jax.ShapeDtypeStruct((NP, HORIZON), _f32),
            jax.ShapeDtypeStruct((NP, F_OUT), _f32),
            jax.ShapeDtypeStruct((NP, F_OUT), _f32),
        ],
    )(x_pad, acc_pp, deg_pp.reshape(2, NP, 1), w0c, w1c, bc, wco, wl, bl)


def kernel(x, edge_index, edge_weight, params):
    xs = jnp.squeeze(x, axis=1)
    x_pad = jnp.pad(xs, ((0, NP - N), (0, 0)))
    src = edge_index[0]
    dst = edge_index[1]

    deg_pp = _deg_kernel(src, dst, edge_weight)
    y = _make_y(deg_pp, x_pad)
    acc_pp = _prop_kernel(src, dst, edge_weight, y)

    p = params
    w0c = jnp.concatenate([p["Wx0_i"], p["Wx0_c"], p["Wx0_o"]], axis=1)
    w1c = jnp.concatenate([p["Wx1_i"], p["Wx1_c"], p["Wx1_o"]], axis=1)
    bc = jnp.concatenate(
        [(p[f"bx_{g}"] + p[f"bh_{g}"])[None, :] + p[f"b_{g}"]
         for g in ("i", "c", "o")], axis=1)
    h, hh, cc = _make_dense(x_pad, acc_pp, deg_pp, w0c, w1c, bc,
                            p["w_c_o"], p["W_lin"], p["b_lin"][None, :])
    return (h[:N], hh[:N], cc[:N])


# speedup vs baseline: 43.3470x; 1.2768x over previous
import functools

import jax
import jax.numpy as jnp
from jax import lax
from jax.experimental import pallas as pl
from jax.experimental.pallas import tpu as pltpu
from jax.experimental.pallas import tpu_sc as plsc

N = 100000
E = 1600000
F_IN = 16
F_OUT = 32
HORIZON = 12

NP = 102400
NW = 32
NPS = NP // 16
EPW = E // NW
CH = 2000
NCHUNK = EPW // CH
GROUPS = CH // 16
CHP = 400
NCHUNKP = EPW // CHP
GROUPSP = CHP // 16

_f32 = jnp.float32
_i32 = jnp.int32

_mesh = plsc.VectorSubcoreMesh(core_axis_name="c", subcore_axis_name="s")


CHD = 10000
NCHUNKD = EPW // CHD
GROUPSD = CHD // 16


@functools.partial(
    pl.kernel,
    out_type=jax.ShapeDtypeStruct((2, NP), _f32),
    mesh=_mesh,
    scratch_types=[
        pltpu.VMEM((3, CHD), _i32),
        pltpu.VMEM((3, CHD), _f32),
        pltpu.VMEM((2, CHD), _i32),
        pltpu.VMEM((2, CHD), _f32),
        pltpu.VMEM((NPS,), _f32),
        pltpu.VMEM_SHARED((NP,), _f32),
        pltpu.SemaphoreType.DMA,
        pltpu.SemaphoreType.DMA,
    ],
    compiler_params=pltpu.CompilerParams(use_tc_tiling_on_sc=False),
)
def _deg_kernel(src_hbm, dst_hbm, ew_hbm, out_hbm,
                src_b, w_b, dst_b, ew_b, zbuf, deg_sh, lsem, ssem):
    cid = lax.axis_index("c")
    sid = lax.axis_index("s")
    wid = cid * 16 + sid
    rbase = sid * NPS

    zero16 = jnp.zeros((16,), _f32)

    def _z(i, _):
        zbuf[pl.ds(i * 16, 16)] = zero16
        return 0

    lax.fori_loop(0, NPS // 16, _z, 0, unroll=8)
    pltpu.sync_copy(zbuf, deg_sh.at[pl.ds(rbase, NPS)])
    plsc.subcore_barrier()

    ebase = wid * EPW

    def _issue_linear(ci):
        off = ebase + ci * CHD
        pltpu.async_copy(src_hbm.at[pl.ds(off, CHD)], src_b.at[ci % 3], lsem)
        pltpu.async_copy(dst_hbm.at[pl.ds(off, CHD)], dst_b.at[ci % 2], lsem)
        pltpu.async_copy(ew_hbm.at[pl.ds(off, CHD)], ew_b.at[ci % 2], lsem)

    def _wait_linear(ci):
        pltpu.make_async_copy(src_hbm.at[pl.ds(0, CHD)], src_b.at[ci % 3], lsem).wait()
        pltpu.make_async_copy(dst_hbm.at[pl.ds(0, CHD)], dst_b.at[ci % 2], lsem).wait()
        pltpu.make_async_copy(ew_hbm.at[pl.ds(0, CHD)], ew_b.at[ci % 2], lsem).wait()

    def _drain_scatter(ci):
        pltpu.make_async_copy(ew_hbm.at[pl.ds(0, CHD)], w_b.at[ci % 3], ssem).wait()

    _issue_linear(0)
    _wait_linear(0)
    for ci in range(NCHUNKD):
        if ci >= 2:
            _drain_scatter(ci - 2)
        if ci + 1 < NCHUNKD:
            _issue_linear(ci + 1)

        def _grp(g, _, _ci=ci):
            s = src_b[_ci % 3, pl.ds(g * 16, 16)]
            d = dst_b[_ci % 2, pl.ds(g * 16, 16)]
            e = ew_b[_ci % 2, pl.ds(g * 16, 16)]
            w_b[_ci % 3, pl.ds(g * 16, 16)] = jnp.where(s == d, 0.0, e)
            return 0

        lax.fori_loop(0, GROUPSD, _grp, 0, unroll=4)
        pltpu.async_copy(w_b.at[ci % 3], deg_sh.at[src_b.at[ci % 3]], ssem,
                         add=True)
        if ci + 1 < NCHUNKD:
            _wait_linear(ci + 1)
    _drain_scatter(NCHUNKD - 2)
    _drain_scatter(NCHUNKD - 1)

    plsc.subcore_barrier()
    pltpu.sync_copy(deg_sh.at[pl.ds(rbase, NPS)],
                    out_hbm.at[cid, pl.ds(rbase, NPS)])


@functools.partial(
    pl.kernel,
    out_type=jax.ShapeDtypeStruct((2, NP, F_IN), _f32),
    mesh=_mesh,
    scratch_types=[
        pltpu.VMEM((2, CHP), _i32),
        pltpu.VMEM((2, CHP), _i32),
        pltpu.VMEM((2, CHP), _f32),
        pltpu.VMEM((CHP,), _f32),
        pltpu.VMEM((2, CHP, F_IN), _f32),
        pltpu.VMEM_SHARED((NP, F_IN), _f32),
        pltpu.SemaphoreType.DMA,
        pltpu.SemaphoreType.DMA,
    ],
    compiler_params=pltpu.CompilerParams(use_tc_tiling_on_sc=False),
)
def _prop_kernel(src_hbm, dst_hbm, ew_hbm, y_hbm, out_hbm,
                 src_b, dst_b, ew_b, w_b, rows, acc_sh, gsem, lsem):
    cid = lax.axis_index("c")
    sid = lax.axis_index("s")
    wid = cid * 16 + sid
    rbase = sid * NPS

    zero16 = jnp.zeros((16,), _f32)

    def _z(i, _):
        rows[0, i, :] = zero16
        return 0

    lax.fori_loop(0, CHP, _z, 0, unroll=8)
    for j in range(NPS // CHP):
        pltpu.sync_copy(rows.at[0], acc_sh.at[pl.ds(rbase + j * CHP, CHP)])
    plsc.subcore_barrier()

    ebase = wid * EPW

    def _issue_linear(ci, b):
        off = ebase + ci * CHP
        pltpu.async_copy(src_hbm.at[pl.ds(off, CHP)], src_b.at[b], lsem)
        pltpu.async_copy(dst_hbm.at[pl.ds(off, CHP)], dst_b.at[b], lsem)
        pltpu.async_copy(ew_hbm.at[pl.ds(off, CHP)], ew_b.at[b], lsem)

    def _wait_linear(b):
        pltpu.make_async_copy(src_hbm.at[pl.ds(0, CHP)], src_b.at[b], lsem).wait()
        pltpu.make_async_copy(dst_hbm.at[pl.ds(0, CHP)], dst_b.at[b], lsem).wait()
        pltpu.make_async_copy(ew_hbm.at[pl.ds(0, CHP)], ew_b.at[b], lsem).wait()

    def _wait_gather(b):
        pltpu.make_async_copy(y_hbm.at[pl.ds(0, CHP)], rows.at[b], gsem).wait()

    def _body(ci, b, last):
        nb = 1 - b

        def _grp(g, _):
            s = src_b[b, pl.ds(g * 16, 16)]
            d = dst_b[b, pl.ds(g * 16, 16)]
            e = ew_b[b, pl.ds(g * 16, 16)]
            w_b[pl.ds(g * 16, 16)] = jnp.where(s == d, 0.0, -e)
            return 0

        if not last:
            _issue_linear(ci + 1, nb)
        lax.fori_loop(0, GROUPSP, _grp, 0, unroll=4)
        _wait_gather(b)

        def _scale(g, _):
            w16 = w_b[pl.ds(g * 16, 16)]
            for j in range(16):
                wv = jnp.full((16,), w16[j], dtype=_f32)
                rows[b, g * 16 + j, :] = rows[b, g * 16 + j, :] * wv
            return 0

        lax.fori_loop(0, GROUPSP, _scale, 0, unroll=2)
        if not last:
            _wait_linear(nb)
            pltpu.async_copy(y_hbm.at[src_b.at[nb]], rows.at[nb], gsem)
        pltpu.sync_copy(rows.at[b], acc_sh.at[dst_b.at[b]], add=True)

    _issue_linear(0, 0)
    _wait_linear(0)
    pltpu.async_copy(y_hbm.at[src_b.at[0]], rows.at[0], gsem)

    def _pair(t, _):
        _body(2 * t, 0, last=False)
        _body(2 * t + 1, 1, last=False)
        return 0

    lax.fori_loop(0, (NCHUNKP - 1) // 2, _pair, 0)
    _body(NCHUNKP - 1, 0, last=True)

    plsc.subcore_barrier()
    pltpu.sync_copy(acc_sh.at[pl.ds(rbase, NPS)],
                    out_hbm.at[cid, pl.ds(rbase, NPS)])


_RB = 4096
_GRID = NP // _RB


def _y_body(deg_ref, x_ref, y_ref, disr_ref):
    deg = deg_ref[0] + deg_ref[1]
    dis = jnp.where(deg > 0, lax.rsqrt(deg), 0.0)
    disr = jnp.broadcast_to(dis, (_RB, F_IN))
    disr_ref[...] = disr
    y_ref[...] = disr * x_ref[...]


def _make_y(deg_pp, x_pad):
    return pl.pallas_call(
        _y_body,
        grid=(_GRID,),
        in_specs=[
            pl.BlockSpec((2, _RB, 1), lambda i: (0, i, 0)),
            pl.BlockSpec((_RB, F_IN), lambda i: (i, 0)),
        ],
        out_specs=[
            pl.BlockSpec((_RB, F_IN), lambda i: (i, 0)),
            pl.BlockSpec((_RB, F_IN), lambda i: (i, 0)),
        ],
        out_shape=[
            jax.ShapeDtypeStruct((NP, F_IN), _f32),
            jax.ShapeDtypeStruct((NP, F_IN), _f32),
        ],
    )(deg_pp.reshape(2, NP, 1), x_pad)


def _sigmoid(z):
    return 0.5 + 0.5 * jnp.tanh(0.5 * z)


def _dense_body(x_ref, a_ref, disr_ref, w0_ref, w1_ref, bc_ref, wco_ref,
                wl_ref, bl_ref, h_ref, hh_ref, cc_ref):
    x = x_ref[...]
    px = disr_ref[...] * (a_ref[0] + a_ref[1])
    pre = (jnp.dot(x, w0_ref[...], preferred_element_type=_f32)
           + jnp.dot(px, w1_ref[...], preferred_element_type=_f32)
           + bc_ref[...])
    gi = _sigmoid(pre[:, 0:F_OUT])
    gt = jnp.tanh(pre[:, F_OUT:2 * F_OUT])
    cc = gi * gt
    go = _sigmoid(pre[:, 2 * F_OUT:3 * F_OUT] + wco_ref[...] * cc)
    hh = go * jnp.tanh(cc)
    h_ref[...] = (jnp.dot(jax.nn.relu(hh), wl_ref[...],
                          preferred_element_type=_f32) + bl_ref[...])
    hh_ref[...] = hh
    cc_ref[...] = cc


def _make_dense(x_pad, acc_pp, disr, w0c, w1c, bc, wco, wl, bl):
    return pl.pallas_call(
        _dense_body,
        grid=(_GRID,),
        in_specs=[
            pl.BlockSpec((_RB, F_IN), lambda i: (i, 0)),
            pl.BlockSpec((2, _RB, F_IN), lambda i: (0, i, 0)),
            pl.BlockSpec((_RB, F_IN), lambda i: (i, 0)),
            pl.BlockSpec((F_IN, 3 * F_OUT), lambda i: (0, 0)),
            pl.BlockSpec((F_IN, 3 * F_OUT), lambda i: (0, 0)),
            pl.BlockSpec((1, 3 * F_OUT), lambda i: (0, 0)),
            pl.BlockSpec((1, F_OUT), lambda i: (0, 0)),
            pl.BlockSpec((F_OUT, HORIZON), lambda i: (0, 0)),
            pl.BlockSpec((1, HORIZON), lambda i: (0, 0)),
        ],
        out_specs=[
            pl.BlockSpec((_RB, HORIZON), lambda i: (i, 0)),
            pl.BlockSpec((_RB, F_OUT), lambda i: (i, 0)),
            pl.BlockSpec((_RB, F_OUT), lambda i: (i, 0)),
        ],
        out_shape=[
            jax.ShapeDtypeStruct((NP, HORIZON), _f32),
            jax.ShapeDtypeStruct((NP, F_OUT), _f32),
            jax.ShapeDtypeStruct((NP, F_OUT), _f32),
        ],
    )(x_pad, acc_pp, disr, w0c, w1c, bc, wco, wl, bl)


def kernel(x, edge_index, edge_weight, params):
    xs = jnp.squeeze(x, axis=1)
    x_pad = jnp.pad(xs, ((0, NP - N), (0, 0)))
    src = edge_index[0]
    dst = edge_index[1]

    deg_pp = _deg_kernel(src, dst, edge_weight)
    y, disr = _make_y(deg_pp, x_pad)
    acc_pp = _prop_kernel(src, dst, edge_weight, y)

    p = params
    w0c = jnp.concatenate([p["Wx0_i"], p["Wx0_c"], p["Wx0_o"]], axis=1)
    w1c = jnp.concatenate([p["Wx1_i"], p["Wx1_c"], p["Wx1_o"]], axis=1)
    bc = jnp.concatenate(
        [(p[f"bx_{g}"] + p[f"bh_{g}"])[None, :] + p[f"b_{g}"]
         for g in ("i", "c", "o")], axis=1)
    h, hh, cc = _make_dense(x_pad, acc_pp, disr, w0c, w1c, bc,
                            p["w_c_o"], p["W_lin"], p["b_lin"][None, :])
    return (h[:N], hh[:N], cc[:N])


# speedup vs baseline: 69.6922x; 1.6078x over previous
import functools

import jax
import jax.numpy as jnp
from jax import lax
from jax.experimental import pallas as pl
from jax.experimental.pallas import tpu as pltpu
from jax.experimental.pallas import tpu_sc as plsc

N = 100000
E = 1600000
F_IN = 16
F_OUT = 32
HORIZON = 12

NP = 102400
NW = 32
NPS = NP // 16
EPW = E // NW
CH = 2000
NCHUNK = EPW // CH
GROUPS = CH // 16
CHP = 400
NCHUNKP = EPW // CHP
GROUPSP = CHP // 16

_f32 = jnp.float32
_i32 = jnp.int32

_mesh = plsc.VectorSubcoreMesh(core_axis_name="c", subcore_axis_name="s")


CHD = 10000
NCHUNKD = EPW // CHD
GROUPSD = CHD // 16


@functools.partial(
    pl.kernel,
    out_type=jax.ShapeDtypeStruct((2, NP, F_IN), _f32),
    mesh=_mesh,
    scratch_types=[
        pltpu.VMEM((3, CHD), _i32),
        pltpu.VMEM((3, CHD), _f32),
        pltpu.VMEM((2, CHD), _i32),
        pltpu.VMEM((2, CHD), _f32),
        pltpu.VMEM((NPS,), _f32),
        pltpu.VMEM((800, F_IN), _f32),
        pltpu.VMEM_SHARED((NP,), _f32),
        pltpu.SemaphoreType.DMA,
        pltpu.SemaphoreType.DMA,
    ],
    compiler_params=pltpu.CompilerParams(use_tc_tiling_on_sc=False),
)
def _deg_kernel(src_hbm, dst_hbm, ew_hbm, out_hbm,
                src_b, w_b, dst_b, ew_b, zbuf, xstage, deg_sh, lsem, ssem):
    cid = lax.axis_index("c")
    sid = lax.axis_index("s")
    wid = cid * 16 + sid
    rbase = sid * NPS

    zero16 = jnp.zeros((16,), _f32)

    def _z(i, _):
        zbuf[pl.ds(i * 16, 16)] = zero16
        return 0

    lax.fori_loop(0, NPS // 16, _z, 0, unroll=8)
    pltpu.sync_copy(zbuf, deg_sh.at[pl.ds(rbase, NPS)])
    plsc.subcore_barrier()

    ebase = wid * EPW

    def _issue_linear(ci):
        off = ebase + ci * CHD
        pltpu.async_copy(src_hbm.at[pl.ds(off, CHD)], src_b.at[ci % 3], lsem)
        pltpu.async_copy(dst_hbm.at[pl.ds(off, CHD)], dst_b.at[ci % 2], lsem)
        pltpu.async_copy(ew_hbm.at[pl.ds(off, CHD)], ew_b.at[ci % 2], lsem)

    def _wait_linear(ci):
        pltpu.make_async_copy(src_hbm.at[pl.ds(0, CHD)], src_b.at[ci % 3], lsem).wait()
        pltpu.make_async_copy(dst_hbm.at[pl.ds(0, CHD)], dst_b.at[ci % 2], lsem).wait()
        pltpu.make_async_copy(ew_hbm.at[pl.ds(0, CHD)], ew_b.at[ci % 2], lsem).wait()

    def _drain_scatter(ci):
        pltpu.make_async_copy(ew_hbm.at[pl.ds(0, CHD)], w_b.at[ci % 3], ssem).wait()

    _issue_linear(0)
    _wait_linear(0)
    for ci in range(NCHUNKD):
        if ci >= 2:
            _drain_scatter(ci - 2)
        if ci + 1 < NCHUNKD:
            _issue_linear(ci + 1)

        def _grp(g, _, _ci=ci):
            s = src_b[_ci % 3, pl.ds(g * 16, 16)]
            d = dst_b[_ci % 2, pl.ds(g * 16, 16)]
            e = ew_b[_ci % 2, pl.ds(g * 16, 16)]
            w_b[_ci % 3, pl.ds(g * 16, 16)] = jnp.where(s == d, 0.0, e)
            return 0

        lax.fori_loop(0, GROUPSD, _grp, 0, unroll=4)
        pltpu.async_copy(w_b.at[ci % 3], deg_sh.at[src_b.at[ci % 3]], ssem,
                         add=True)
        if ci + 1 < NCHUNKD:
            _wait_linear(ci + 1)
    _drain_scatter(NCHUNKD - 2)
    _drain_scatter(NCHUNKD - 1)

    plsc.subcore_barrier()
    pltpu.sync_copy(deg_sh.at[pl.ds(rbase, NPS)], zbuf)
    for c in range(NPS // 800):
        def _exp(g, _, _c=c):
            d16 = zbuf[pl.ds(_c * 800 + g * 16, 16)]
            for j in range(16):
                xstage[g * 16 + j, :] = jnp.full((16,), d16[j], dtype=_f32)
            return 0

        lax.fori_loop(0, 50, _exp, 0)
        pltpu.sync_copy(xstage,
                        out_hbm.at[cid, pl.ds(rbase + c * 800, 800)])


@functools.partial(
    pl.kernel,
    out_type=jax.ShapeDtypeStruct((2, NP, F_IN), _f32),
    mesh=_mesh,
    scratch_types=[
        pltpu.VMEM((2, CHP), _i32),
        pltpu.VMEM((2, CHP), _i32),
        pltpu.VMEM((2, CHP), _f32),
        pltpu.VMEM((CHP,), _f32),
        pltpu.VMEM((2, CHP, F_IN), _f32),
        pltpu.VMEM_SHARED((NP, F_IN), _f32),
        pltpu.SemaphoreType.DMA,
        pltpu.SemaphoreType.DMA,
    ],
    compiler_params=pltpu.CompilerParams(use_tc_tiling_on_sc=False),
)
def _prop_kernel(src_hbm, dst_hbm, ew_hbm, y_hbm, out_hbm,
                 src_b, dst_b, ew_b, w_b, rows, acc_sh, gsem, lsem):
    cid = lax.axis_index("c")
    sid = lax.axis_index("s")
    wid = cid * 16 + sid
    rbase = sid * NPS

    zero16 = jnp.zeros((16,), _f32)

    def _z(i, _):
        rows[0, i, :] = zero16
        return 0

    lax.fori_loop(0, CHP, _z, 0, unroll=8)
    for j in range(NPS // CHP):
        pltpu.sync_copy(rows.at[0], acc_sh.at[pl.ds(rbase + j * CHP, CHP)])
    plsc.subcore_barrier()

    ebase = wid * EPW

    def _issue_linear(ci, b):
        off = ebase + ci * CHP
        pltpu.async_copy(src_hbm.at[pl.ds(off, CHP)], src_b.at[b], lsem)
        pltpu.async_copy(dst_hbm.at[pl.ds(off, CHP)], dst_b.at[b], lsem)
        pltpu.async_copy(ew_hbm.at[pl.ds(off, CHP)], ew_b.at[b], lsem)

    def _wait_linear(b):
        pltpu.make_async_copy(src_hbm.at[pl.ds(0, CHP)], src_b.at[b], lsem).wait()
        pltpu.make_async_copy(dst_hbm.at[pl.ds(0, CHP)], dst_b.at[b], lsem).wait()
        pltpu.make_async_copy(ew_hbm.at[pl.ds(0, CHP)], ew_b.at[b], lsem).wait()

    def _wait_gather(b):
        pltpu.make_async_copy(y_hbm.at[pl.ds(0, CHP)], rows.at[b], gsem).wait()

    def _body(ci, b, last):
        nb = 1 - b

        def _grp(g, _):
            s = src_b[b, pl.ds(g * 16, 16)]
            d = dst_b[b, pl.ds(g * 16, 16)]
            e = ew_b[b, pl.ds(g * 16, 16)]
            w_b[pl.ds(g * 16, 16)] = jnp.where(s == d, 0.0, -e)
            return 0

        if not last:
            _issue_linear(ci + 1, nb)
        lax.fori_loop(0, GROUPSP, _grp, 0, unroll=4)
        _wait_gather(b)

        def _scale(g, _):
            w16 = w_b[pl.ds(g * 16, 16)]
            for j in range(16):
                wv = jnp.full((16,), w16[j], dtype=_f32)
                rows[b, g * 16 + j, :] = rows[b, g * 16 + j, :] * wv
            return 0

        lax.fori_loop(0, GROUPSP, _scale, 0, unroll=2)
        if not last:
            _wait_linear(nb)
            pltpu.async_copy(y_hbm.at[src_b.at[nb]], rows.at[nb], gsem)
        pltpu.sync_copy(rows.at[b], acc_sh.at[dst_b.at[b]], add=True)

    _issue_linear(0, 0)
    _wait_linear(0)
    pltpu.async_copy(y_hbm.at[src_b.at[0]], rows.at[0], gsem)

    def _pair(t, _):
        _body(2 * t, 0, last=False)
        _body(2 * t + 1, 1, last=False)
        return 0

    lax.fori_loop(0, (NCHUNKP - 1) // 2, _pair, 0)
    _body(NCHUNKP - 1, 0, last=True)

    plsc.subcore_barrier()
    pltpu.sync_copy(acc_sh.at[pl.ds(rbase, NPS)],
                    out_hbm.at[cid, pl.ds(rbase, NPS)])


_NR = NP // 8
_RB = 512
_GRID = _NR // _RB


def _y_body(degx_ref, x_ref, y_ref):
    deg = degx_ref[0] + degx_ref[1]
    dis = jnp.where(deg > 0, lax.rsqrt(deg), 0.0)
    y_ref[...] = dis * x_ref[...]


def _make_y(degx_p, x_p):
    return pl.pallas_call(
        _y_body,
        grid=(_GRID,),
        in_specs=[
            pl.BlockSpec((2, _RB, 128), lambda i: (0, i, 0)),
            pl.BlockSpec((_RB, 128), lambda i: (i, 0)),
        ],
        out_specs=pl.BlockSpec((_RB, 128), lambda i: (i, 0)),
        out_shape=jax.ShapeDtypeStruct((_NR, 128), _f32),
    )(degx_p, x_p)


def _sigmoid(z):
    return 0.5 + 0.5 * jnp.tanh(0.5 * z)


def _dense_body(x_ref, a_ref, degx_ref, w0_ref, w1_ref, bc_ref, wco_ref,
                wl_ref, bl_ref, h_ref, hh_ref, cc_ref):
    x = x_ref[...]
    deg = degx_ref[0] + degx_ref[1]
    dis = jnp.where(deg > 0, lax.rsqrt(deg), 0.0)
    px = dis * (a_ref[0] + a_ref[1])
    pre = (jnp.dot(x, w0_ref[...], preferred_element_type=_f32)
           + jnp.dot(px, w1_ref[...], preferred_element_type=_f32)
           + bc_ref[...])
    gi = _sigmoid(pre[:, 0:256])
    gt = jnp.tanh(pre[:, 256:512])
    cc = gi * gt
    go = _sigmoid(pre[:, 512:768] + wco_ref[...] * cc)
    hh = go * jnp.tanh(cc)
    h_ref[...] = (jnp.dot(jax.nn.relu(hh), wl_ref[...],
                          preferred_element_type=_f32) + bl_ref[...])
    hh_ref[...] = hh
    cc_ref[...] = cc


def _make_dense(x_p, acc_p, degx_p, w0_blk, w1_blk, bc_blk, wco_blk,
                wl_blk, bl_blk):
    return pl.pallas_call(
        _dense_body,
        grid=(_GRID,),
        in_specs=[
            pl.BlockSpec((_RB, 128), lambda i: (i, 0)),
            pl.BlockSpec((2, _RB, 128), lambda i: (0, i, 0)),
            pl.BlockSpec((2, _RB, 128), lambda i: (0, i, 0)),
            pl.BlockSpec((128, 768), lambda i: (0, 0)),
            pl.BlockSpec((128, 768), lambda i: (0, 0)),
            pl.BlockSpec((1, 768), lambda i: (0, 0)),
            pl.BlockSpec((1, 256), lambda i: (0, 0)),
            pl.BlockSpec((256, 8 * HORIZON), lambda i: (0, 0)),
            pl.BlockSpec((1, 8 * HORIZON), lambda i: (0, 0)),
        ],
        out_specs=[
            pl.BlockSpec((_RB, 8 * HORIZON), lambda i: (i, 0)),
            pl.BlockSpec((_RB, 256), lambda i: (i, 0)),
            pl.BlockSpec((_RB, 256), lambda i: (i, 0)),
        ],
        out_shape=[
            jax.ShapeDtypeStruct((_NR, 8 * HORIZON), _f32),
            jax.ShapeDtypeStruct((_NR, 256), _f32),
            jax.ShapeDtypeStruct((_NR, 256), _f32),
        ],
    )(x_p, acc_p, degx_p, w0_blk, w1_blk, bc_blk, wco_blk, wl_blk, bl_blk)


def _pack_weights(p):
    eye8 = jnp.eye(8, dtype=_f32)
    w0s = jnp.stack([p["Wx0_i"], p["Wx0_c"], p["Wx0_o"]])
    w1s = jnp.stack([p["Wx1_i"], p["Wx1_c"], p["Wx1_o"]])
    w0_blk = jnp.einsum("kK,gfj->kfgKj", eye8, w0s).reshape(128, 768)
    w1_blk = jnp.einsum("kK,gfj->kfgKj", eye8, w1s).reshape(128, 768)
    bs = jnp.stack([(p[f"bx_{g}"] + p[f"bh_{g}"])[None, :] + p[f"b_{g}"]
                    for g in ("i", "c", "o")])
    bc_blk = jnp.broadcast_to(bs[:, 0, None, :], (3, 8, 32)).reshape(1, 768)
    wco_blk = jnp.broadcast_to(p["w_c_o"], (8, 32)).reshape(1, 256)
    wl_blk = jnp.einsum("kK,fj->kfKj", eye8, p["W_lin"]).reshape(256, 96)
    bl_blk = jnp.broadcast_to(p["b_lin"][None, :], (8, HORIZON)).reshape(1, 96)
    return w0_blk, w1_blk, bc_blk, wco_blk, wl_blk, bl_blk


def kernel(x, edge_index, edge_weight, params):
    x_p8 = jnp.reshape(x, (N // 8, 128))
    x_p = jnp.pad(x_p8, ((0, (NP - N) // 8), (0, 0)))
    src = edge_index[0]
    dst = edge_index[1]

    degx = _deg_kernel(src, dst, edge_weight)
    degx_p = degx.reshape(2, _NR, 128)
    y_p = _make_y(degx_p, x_p)
    acc = _prop_kernel(src, dst, edge_weight, y_p.reshape(NP, F_IN))
    acc_p = acc.reshape(2, _NR, 128)

    w0_blk, w1_blk, bc_blk, wco_blk, wl_blk, bl_blk = _pack_weights(params)
    h_p, hh_p, cc_p = _make_dense(x_p, acc_p, degx_p, w0_blk, w1_blk,
                                  bc_blk, wco_blk, wl_blk, bl_blk)
    h = h_p[:N // 8].reshape(N, HORIZON)
    hh = hh_p[:N // 8].reshape(N, F_OUT)
    cc = cc_p[:N // 8].reshape(N, F_OUT)
    return (h, hh, cc)
